# baseline (speedup 1.0000x reference)
import functools

import jax
import jax.numpy as jnp
from jax import lax
from jax.experimental import pallas as pl
from jax.experimental.pallas import tpu as pltpu
from jax.experimental.pallas import tpu_sc as plsc

ALPHA = 0.05
N_TOKENS = 16384
INPUT_SIZE = 128
K = 8192
DIM = 2 * INPUT_SIZE

NW = 32
TPW = N_TOKENS // NW
CHUNK = 128
NCHUNK = TPW // CHUNK
L = 16
NGRP = CHUNK // L

_STATS_BLK = 1024
_STATS_GRID = K // _STATS_BLK


def _stats_body(t_ref, o_ref):
    i = pl.program_id(0)
    blk = t_ref[...]
    ps = jnp.sum(blk, axis=0, keepdims=True) * (-2.0 / K)
    pq = jnp.sum(blk * blk) * (1.0 / K)

    @pl.when(i == 0)
    def _():
        o_ref[...] = jnp.zeros((2, DIM), jnp.float32)

    o_ref[0:1, :] += ps
    o_ref[1:2, :] += jnp.full((1, DIM), pq, jnp.float32)


_stats_call = pl.pallas_call(
    _stats_body,
    grid=(_STATS_GRID,),
    in_specs=[pl.BlockSpec((_STATS_BLK, DIM), lambda i: (i, 0))],
    out_specs=pl.BlockSpec((2, DIM), lambda i: (0, 0)),
    out_shape=jax.ShapeDtypeStruct((2, DIM), jnp.float32),
)

_mesh = plsc.VectorSubcoreMesh(core_axis_name="c", subcore_axis_name="s")


@functools.partial(
    pl.kernel,
    mesh=_mesh,
    compiler_params=pltpu.CompilerParams(needs_layout_passes=False),
    out_type=jax.ShapeDtypeStruct((N_TOKENS, 4), jnp.float32),
    scratch_types=[
        pltpu.VMEM((CHUNK,), jnp.int32),
        pltpu.VMEM((CHUNK, DIM), jnp.float32),
        pltpu.VMEM((CHUNK, INPUT_SIZE), jnp.float32),
        pltpu.VMEM((CHUNK, 4), jnp.float32),
        pltpu.VMEM((2 * DIM,), jnp.float32),
        pltpu.SemaphoreType.DMA,
    ],
)
def _sc_art(x_hbm, ms_hbm, tpl_hbm, stats_hbm, out_hbm,
            idx_v, y_v, x_v, o_v, st_v, sem):
    wid = lax.axis_index("s") * 2 + lax.axis_index("c")
    base = wid * TPW
    pltpu.sync_copy(stats_hbm, st_v)
    c0v = st_v[pl.ds(DIM, L)]
    iota = lax.iota(jnp.int32, L)
    zero = jnp.zeros((L,), jnp.float32)

    def chunk_body(c, carry):
        cbase = base + c * CHUNK
        pltpu.sync_copy(ms_hbm.at[pl.ds(cbase, CHUNK)], idx_v)
        gat = pltpu.async_copy(tpl_hbm.at[idx_v], y_v, sem)
        pltpu.sync_copy(x_hbm.at[pl.ds(cbase, CHUNK)], x_v)
        gat.wait()

        def grp_body(g, carry2):
            rows = g * L + iota

            def dim_body(d, accs):
                acc_d, acc_a, acc_w, acc_ys = accs
                cols = jnp.full((L,), d, jnp.int32)
                colsb = cols + INPUT_SIZE
                xv = plsc.load_gather(x_v, [rows, cols])
                ya = plsc.load_gather(y_v, [rows, cols])
                yb = plsc.load_gather(y_v, [rows, colsb])
                ta = plsc.load_gather(st_v, [cols])
                tb = plsc.load_gather(st_v, [colsb])
                s = 1.0 / (1.0 + jnp.exp(-xv))
                cmp = 1.0 - s
                da = s - ya
                db = cmp - yb
                acc_d = acc_d + da * da + db * db
                acc_a = acc_a + jnp.minimum(s, ya) + jnp.minimum(cmp, yb)
                acc_w = acc_w + ya * (ya + ta) + yb * (yb + tb)
                acc_ys = acc_ys + ya + yb
                return acc_d, acc_a, acc_w, acc_ys

            acc_d, acc_a, acc_w, acc_ys = lax.fori_loop(
                0, INPUT_SIZE, dim_body, (zero, zero, c0v, zero))
            sam = acc_a / (ALPHA + acc_ys)
            res = acc_a * (1.0 / INPUT_SIZE)
            plsc.store_scatter(o_v, [rows, jnp.full((L,), 0, jnp.int32)], acc_d)
            plsc.store_scatter(o_v, [rows, jnp.full((L,), 1, jnp.int32)], acc_w)
            plsc.store_scatter(o_v, [rows, jnp.full((L,), 2, jnp.int32)], sam)
            plsc.store_scatter(o_v, [rows, jnp.full((L,), 3, jnp.int32)], res)
            return carry2

        lax.fori_loop(0, NGRP, grp_body, 0)
        pltpu.sync_copy(o_v, out_hbm.at[pl.ds(cbase, CHUNK)])
        return carry

    lax.fori_loop(0, NCHUNK, chunk_body, 0)


@jax.jit
def kernel(x, ms, templates):
    stats = _stats_call(templates).reshape(2 * DIM)
    return _sc_art(x, ms.astype(jnp.int32), templates, stats)


# speedup vs baseline: 1.8221x; 1.8221x over previous
import functools

import jax
import jax.numpy as jnp
from jax import lax
from jax.experimental import pallas as pl
from jax.experimental.pallas import tpu as pltpu
from jax.experimental.pallas import tpu_sc as plsc

ALPHA = 0.05
N_TOKENS = 16384
INPUT_SIZE = 128
K = 8192
DIM = 2 * INPUT_SIZE

NW = 32
TPW = N_TOKENS // NW
CHUNK = 128
NCHUNK = TPW // CHUNK
L = 16
NGRP = CHUNK // L

_BLK = 1024
_GRID = K // _BLK


def _stats_body(t_ref, o_ref):
    i = pl.program_id(0)
    blk = t_ref[...]
    ps = jnp.sum(blk, axis=0, keepdims=True) * (-2.0 / K)
    pq = jnp.sum(blk * blk) * (1.0 / K)

    @pl.when(i == 0)
    def _():
        o_ref[...] = jnp.zeros((2, DIM), jnp.float32)

    o_ref[0:1, :] += ps
    o_ref[1:2, :] += jnp.full((1, DIM), pq, jnp.float32)


_stats_call = pl.pallas_call(
    _stats_body,
    grid=(_GRID,),
    in_specs=[pl.BlockSpec((_BLK, DIM), lambda i: (i, 0))],
    out_specs=pl.BlockSpec((2, DIM), lambda i: (0, 0)),
    out_shape=jax.ShapeDtypeStruct((2, DIM), jnp.float32),
)


def _tbl_body(t_ref, st_ref, dwo_ref, sden_ref):
    blk = t_ref[...]
    tsn = st_ref[0:1, :]
    c0 = st_ref[1, 0]
    s2sv = jnp.sum(blk * blk + blk * tsn, axis=1)
    s1 = jnp.sum(blk, axis=1)
    dwo_ref[...] = (c0 + s2sv).reshape(_BLK // INPUT_SIZE, INPUT_SIZE)
    sden_ref[...] = (ALPHA + s1).reshape(_BLK // INPUT_SIZE, INPUT_SIZE)


_tbl_call = pl.pallas_call(
    _tbl_body,
    grid=(_GRID,),
    in_specs=[
        pl.BlockSpec((_BLK, DIM), lambda i: (i, 0)),
        pl.BlockSpec((2, DIM), lambda i: (0, 0)),
    ],
    out_specs=[
        pl.BlockSpec((_BLK // INPUT_SIZE, INPUT_SIZE), lambda i: (i, 0)),
        pl.BlockSpec((_BLK // INPUT_SIZE, INPUT_SIZE), lambda i: (i, 0)),
    ],
    out_shape=[
        jax.ShapeDtypeStruct((K // INPUT_SIZE, INPUT_SIZE), jnp.float32),
        jax.ShapeDtypeStruct((K // INPUT_SIZE, INPUT_SIZE), jnp.float32),
    ],
)

_mesh = plsc.VectorSubcoreMesh(core_axis_name="c", subcore_axis_name="s")


@functools.partial(
    pl.kernel,
    mesh=_mesh,
    compiler_params=pltpu.CompilerParams(needs_layout_passes=False),
    out_type=jax.ShapeDtypeStruct((N_TOKENS, 4), jnp.float32),
    scratch_types=[
        pltpu.VMEM((CHUNK,), jnp.int32),
        pltpu.VMEM((CHUNK,), jnp.int32),
        pltpu.VMEM((CHUNK, DIM), jnp.float32),
        pltpu.VMEM((CHUNK, DIM), jnp.float32),
        pltpu.VMEM((CHUNK, INPUT_SIZE), jnp.float32),
        pltpu.VMEM((CHUNK, 4), jnp.float32),
        pltpu.VMEM((K,), jnp.float32),
        pltpu.VMEM((K,), jnp.float32),
        pltpu.SemaphoreType.DMA,
        pltpu.SemaphoreType.DMA,
    ],
)
def _sc_art(x_hbm, ms_hbm, tpl_hbm, dwo_hbm, sden_hbm, out_hbm,
            idx0, idx1, y0, y1, x_v, o_v, dwo_v, sden_v,
            sy0, sy1):
    wid = lax.axis_index("s") * 2 + lax.axis_index("c")
    base = wid * TPW
    pltpu.sync_copy(dwo_hbm, dwo_v)
    pltpu.sync_copy(sden_hbm, sden_v)
    iota = lax.iota(jnp.int32, L)
    zero = jnp.zeros((L,), jnp.float32)
    idx = (idx0, idx1)
    yb_ = (y0, y1)
    sy = (sy0, sy1)

    def start_chunk(c):
        b = c & 1
        cbase = base + c * CHUNK
        pltpu.sync_copy(ms_hbm.at[pl.ds(cbase, CHUNK)], idx[b])
        return pltpu.async_copy(tpl_hbm.at[idx[b]], yb_[b], sy[b])

    pend = start_chunk(0)
    for c in range(NCHUNK):
        b = c & 1
        if c + 1 < NCHUNK:
            nxt = start_chunk(c + 1)
        pltpu.sync_copy(x_hbm.at[pl.ds(base + c * CHUNK, CHUNK)], x_v)
        pend.wait()
        y_v = yb_[b]

        def grp_body(g, carry2):
            rows = g * L + iota
            msv = idx[b][pl.ds(g * L, L)]
            dwo_g = plsc.load_gather(dwo_v, [msv])
            sden_g = plsc.load_gather(sden_v, [msv])

            def dim_body(d, accs):
                acc_d, acc_a = accs
                cols = (d + iota) & (INPUT_SIZE - 1)
                xv = plsc.load_gather(x_v, [rows, cols])
                ya = plsc.load_gather(y_v, [rows, cols])
                yc = plsc.load_gather(y_v, [rows, cols + INPUT_SIZE])
                s = 1.0 / (1.0 + jnp.exp(-xv))
                cmp = 1.0 - s
                da = s - ya
                db = cmp - yc
                acc_d = acc_d + da * da + db * db
                acc_a = acc_a + jnp.minimum(s, ya) + jnp.minimum(cmp, yc)
                return acc_d, acc_a

            acc_d, acc_a = lax.fori_loop(
                0, INPUT_SIZE, dim_body, (zero, zero))
            sam = acc_a / sden_g
            res = acc_a * (1.0 / INPUT_SIZE)
            plsc.store_scatter(o_v, [rows, jnp.full((L,), 0, jnp.int32)], acc_d)
            plsc.store_scatter(o_v, [rows, jnp.full((L,), 1, jnp.int32)], dwo_g)
            plsc.store_scatter(o_v, [rows, jnp.full((L,), 2, jnp.int32)], sam)
            plsc.store_scatter(o_v, [rows, jnp.full((L,), 3, jnp.int32)], res)
            return carry2

        lax.fori_loop(0, NGRP, grp_body, 0)
        cbase = base + c * CHUNK
        pltpu.sync_copy(o_v, out_hbm.at[pl.ds(cbase, CHUNK)])
        if c + 1 < NCHUNK:
            pend = nxt


@jax.jit
def kernel(x, ms, templates):
    stats = _stats_call(templates)
    dwo, sden = _tbl_call(templates, stats)
    return _sc_art(x, ms.astype(jnp.int32), templates,
                   dwo.reshape(K), sden.reshape(K))


# speedup vs baseline: 2.3950x; 1.3144x over previous
import functools

import jax
import jax.numpy as jnp
from jax import lax
from jax.experimental import pallas as pl
from jax.experimental.pallas import tpu as pltpu
from jax.experimental.pallas import tpu_sc as plsc

ALPHA = 0.05
N_TOKENS = 16384
INPUT_SIZE = 128
K = 8192
DIM = 2 * INPUT_SIZE

NW = 32
TPW = N_TOKENS // NW
CHUNK = 64
NCHUNK = TPW // CHUNK
L = 16
NGRP = CHUNK // L

_BLK = 1024
_GRID = K // _BLK


def _stats_body(t_ref, o_ref):
    i = pl.program_id(0)
    blk = t_ref[...]
    ps = jnp.sum(blk, axis=0, keepdims=True) * (-2.0 / K)
    pq = jnp.sum(blk * blk) * (1.0 / K)

    @pl.when(i == 0)
    def _():
        o_ref[...] = jnp.zeros((2, DIM), jnp.float32)

    o_ref[0:1, :] += ps
    o_ref[1:2, :] += jnp.full((1, DIM), pq, jnp.float32)


_stats_call = pl.pallas_call(
    _stats_body,
    grid=(_GRID,),
    in_specs=[pl.BlockSpec((_BLK, DIM), lambda i: (i, 0))],
    out_specs=pl.BlockSpec((2, DIM), lambda i: (0, 0)),
    out_shape=jax.ShapeDtypeStruct((2, DIM), jnp.float32),
)


def _tbl_body(t_ref, st_ref, dwo_ref, sden_ref):
    blk = t_ref[...]
    tsn = st_ref[0:1, :]
    c0 = st_ref[1, 0]
    s2sv = jnp.sum(blk * blk + blk * tsn, axis=1)
    s1 = jnp.sum(blk, axis=1)
    dwo_ref[...] = (c0 + s2sv).reshape(_BLK // INPUT_SIZE, INPUT_SIZE)
    sden_ref[...] = (ALPHA + s1).reshape(_BLK // INPUT_SIZE, INPUT_SIZE)


_tbl_call = pl.pallas_call(
    _tbl_body,
    grid=(_GRID,),
    in_specs=[
        pl.BlockSpec((_BLK, DIM), lambda i: (i, 0)),
        pl.BlockSpec((2, DIM), lambda i: (0, 0)),
    ],
    out_specs=[
        pl.BlockSpec((_BLK // INPUT_SIZE, INPUT_SIZE), lambda i: (i, 0)),
        pl.BlockSpec((_BLK // INPUT_SIZE, INPUT_SIZE), lambda i: (i, 0)),
    ],
    out_shape=[
        jax.ShapeDtypeStruct((K // INPUT_SIZE, INPUT_SIZE), jnp.float32),
        jax.ShapeDtypeStruct((K // INPUT_SIZE, INPUT_SIZE), jnp.float32),
    ],
)

_mesh = plsc.VectorSubcoreMesh(core_axis_name="c", subcore_axis_name="s")


@functools.partial(
    pl.kernel,
    mesh=_mesh,
    compiler_params=pltpu.CompilerParams(needs_layout_passes=False),
    out_type=jax.ShapeDtypeStruct((N_TOKENS, 4), jnp.float32),
    scratch_types=[
        pltpu.VMEM((TPW,), jnp.int32),
        pltpu.VMEM((CHUNK, DIM), jnp.float32),
        pltpu.VMEM((CHUNK, DIM), jnp.float32),
        pltpu.VMEM((CHUNK, INPUT_SIZE), jnp.float32),
        pltpu.VMEM((CHUNK, INPUT_SIZE), jnp.float32),
        pltpu.VMEM((CHUNK,), jnp.float32),
        pltpu.VMEM((CHUNK,), jnp.float32),
        pltpu.VMEM((CHUNK,), jnp.float32),
        pltpu.VMEM((CHUNK,), jnp.float32),
        pltpu.VMEM((CHUNK, 4), jnp.float32),
        pltpu.VMEM((CHUNK, 4), jnp.float32),
    ] + [pltpu.SemaphoreType.DMA] * 10,
)
def _sc_art(x_hbm, ms_hbm, tpl_hbm, dwo_hbm, sden_hbm, out_hbm,
            idx_all, y0, y1, x0, x1, dw0, dw1, sd0, sd1, o0, o1,
            sy0, sy1, sx0, sx1, sd0_, sd1_, ss0, ss1, so0, so1):
    wid = lax.axis_index("s") * 2 + lax.axis_index("c")
    base = wid * TPW
    pltpu.sync_copy(ms_hbm.at[pl.ds(base, TPW)], idx_all)
    iota = lax.iota(jnp.int32, L)
    zero = jnp.zeros((L,), jnp.float32)
    yb_ = (y0, y1)
    xb_ = (x0, x1)
    dwc = (dw0, dw1)
    sdc = (sd0, sd1)
    ob_ = (o0, o1)
    sy = (sy0, sy1)
    sx = (sx0, sx1)
    sd = (sd0_, sd1_)
    ss = (ss0, ss1)
    so = (so0, so1)

    def start_chunk(c):
        b = c & 1
        cbase = base + c * CHUNK
        idx_c = idx_all.at[pl.ds(c * CHUNK, CHUNK)]
        return (
            pltpu.async_copy(tpl_hbm.at[idx_c], yb_[b], sy[b]),
            pltpu.async_copy(x_hbm.at[pl.ds(cbase, CHUNK)], xb_[b], sx[b]),
            pltpu.async_copy(dwo_hbm.at[idx_c], dwc[b], sd[b]),
            pltpu.async_copy(sden_hbm.at[idx_c], sdc[b], ss[b]),
        )

    pend = start_chunk(0)
    out_pend = [None, None]
    for c in range(NCHUNK):
        b = c & 1
        if c + 1 < NCHUNK:
            nxt = start_chunk(c + 1)
        for h in pend:
            h.wait()
        if out_pend[b] is not None:
            out_pend[b].wait()
        y_v = yb_[b]
        x_v = xb_[b]
        o_v = ob_[b]

        def grp_body(g, carry2):
            rows = g * L + iota

            def dim_body(d, accs):
                acc_d, acc_a = accs
                cols = (d + iota) & (INPUT_SIZE - 1)
                xv = plsc.load_gather(x_v, [rows, cols])
                ya = plsc.load_gather(y_v, [rows, cols])
                yc = plsc.load_gather(y_v, [rows, cols + INPUT_SIZE])
                s = 1.0 / (1.0 + jnp.exp(-xv))
                cmp = 1.0 - s
                da = s - ya
                db = cmp - yc
                acc_d = acc_d + da * da + db * db
                acc_a = acc_a + jnp.minimum(s, ya) + jnp.minimum(cmp, yc)
                return acc_d, acc_a

            acc_d, acc_a = lax.fori_loop(
                0, INPUT_SIZE, dim_body, (zero, zero), unroll=2)
            dwo_g = dwc[b][pl.ds(g * L, L)]
            sden_g = sdc[b][pl.ds(g * L, L)]
            sam = acc_a / sden_g
            res = acc_a * (1.0 / INPUT_SIZE)
            plsc.store_scatter(o_v, [rows, jnp.full((L,), 0, jnp.int32)], acc_d)
            plsc.store_scatter(o_v, [rows, jnp.full((L,), 1, jnp.int32)], dwo_g)
            plsc.store_scatter(o_v, [rows, jnp.full((L,), 2, jnp.int32)], sam)
            plsc.store_scatter(o_v, [rows, jnp.full((L,), 3, jnp.int32)], res)
            return carry2

        lax.fori_loop(0, NGRP, grp_body, 0)
        out_pend[b] = pltpu.async_copy(
            o_v, out_hbm.at[pl.ds(base + c * CHUNK, CHUNK)], so[b])
        if c + 1 < NCHUNK:
            pend = nxt
    for h in out_pend:
        if h is not None:
            h.wait()


@jax.jit
def kernel(x, ms, templates):
    stats = _stats_call(templates)
    dwo, sden = _tbl_call(templates, stats)
    return _sc_art(x, ms.astype(jnp.int32), templates,
                   dwo.reshape(K), sden.reshape(K))


# speedup vs baseline: 2.7780x; 1.1599x over previous
import functools

import jax
import jax.numpy as jnp
from jax import lax
from jax.experimental import pallas as pl
from jax.experimental.pallas import tpu as pltpu
from jax.experimental.pallas import tpu_sc as plsc

ALPHA = 0.05
N_TOKENS = 16384
INPUT_SIZE = 128
K = 8192
DIM = 2 * INPUT_SIZE

NW = 32
TPW = N_TOKENS // NW
CHUNK = 64
NCHUNK = TPW // CHUNK
L = 16
NGRP = CHUNK // L

_BLK = 1024
_GRID = K // _BLK


def _stats_body(t_ref, o_ref):
    i = pl.program_id(0)
    blk = t_ref[...]
    ps = jnp.sum(blk, axis=0, keepdims=True) * (-2.0 / K)
    pq = jnp.sum(blk * blk) * (1.0 / K)

    @pl.when(i == 0)
    def _():
        o_ref[...] = jnp.zeros((2, DIM), jnp.float32)

    o_ref[0:1, :] += ps
    o_ref[1:2, :] += jnp.full((1, DIM), pq, jnp.float32)


_stats_call = pl.pallas_call(
    _stats_body,
    grid=(_GRID,),
    in_specs=[pl.BlockSpec((_BLK, DIM), lambda i: (i, 0))],
    out_specs=pl.BlockSpec((2, DIM), lambda i: (0, 0)),
    out_shape=jax.ShapeDtypeStruct((2, DIM), jnp.float32),
)


def _tbl_body(t_ref, st_ref, dwo_ref, sden_ref):
    blk = t_ref[...]
    tsn = st_ref[0:1, :]
    c0 = st_ref[1, 0]
    s2sv = jnp.sum(blk * blk + blk * tsn, axis=1)
    s1 = jnp.sum(blk, axis=1)
    dwo_ref[...] = (c0 + s2sv).reshape(_BLK // INPUT_SIZE, INPUT_SIZE)
    sden_ref[...] = (ALPHA + s1).reshape(_BLK // INPUT_SIZE, INPUT_SIZE)


_tbl_call = pl.pallas_call(
    _tbl_body,
    grid=(_GRID,),
    in_specs=[
        pl.BlockSpec((_BLK, DIM), lambda i: (i, 0)),
        pl.BlockSpec((2, DIM), lambda i: (0, 0)),
    ],
    out_specs=[
        pl.BlockSpec((_BLK // INPUT_SIZE, INPUT_SIZE), lambda i: (i, 0)),
        pl.BlockSpec((_BLK // INPUT_SIZE, INPUT_SIZE), lambda i: (i, 0)),
    ],
    out_shape=[
        jax.ShapeDtypeStruct((K // INPUT_SIZE, INPUT_SIZE), jnp.float32),
        jax.ShapeDtypeStruct((K // INPUT_SIZE, INPUT_SIZE), jnp.float32),
    ],
)

_mesh = plsc.VectorSubcoreMesh(core_axis_name="c", subcore_axis_name="s")


@functools.partial(
    pl.kernel,
    mesh=_mesh,
    compiler_params=pltpu.CompilerParams(needs_layout_passes=False),
    out_type=[
        jax.ShapeDtypeStruct((N_TOKENS,), jnp.float32),
        jax.ShapeDtypeStruct((N_TOKENS,), jnp.float32),
        jax.ShapeDtypeStruct((N_TOKENS,), jnp.float32),
    ],
    scratch_types=[
        pltpu.VMEM((TPW,), jnp.int32),
        pltpu.VMEM((CHUNK, DIM), jnp.float32),
        pltpu.VMEM((CHUNK, DIM), jnp.float32),
        pltpu.VMEM((CHUNK, INPUT_SIZE), jnp.float32),
        pltpu.VMEM((CHUNK, INPUT_SIZE), jnp.float32),
        pltpu.VMEM((CHUNK,), jnp.float32),
        pltpu.VMEM((CHUNK,), jnp.float32),
        pltpu.VMEM((CHUNK,), jnp.float32),
        pltpu.VMEM((CHUNK,), jnp.float32),
        pltpu.VMEM((CHUNK,), jnp.float32),
        pltpu.VMEM((CHUNK,), jnp.float32),
    ] + [pltpu.SemaphoreType.DMA] * 10,
)
def _sc_main(x_hbm, ms_hbm, tpl_hbm, d_hbm, a_hbm, r_hbm,
             idx_all, y0, y1, x0, x1, od0, od1, oa0, oa1, or0, or1,
             sy0, sy1, sx0, sx1, sd0, sd1, sa0, sa1, sr0, sr1):
    wid = lax.axis_index("s") * 2 + lax.axis_index("c")
    base = wid * TPW
    pltpu.sync_copy(ms_hbm.at[pl.ds(base, TPW)], idx_all)
    iota = lax.iota(jnp.int32, L)
    zero = jnp.zeros((L,), jnp.float32)
    yb_ = (y0, y1)
    xb_ = (x0, x1)
    od = (od0, od1)
    oa = (oa0, oa1)
    orr = (or0, or1)
    sy = (sy0, sy1)
    sx = (sx0, sx1)
    sdh = (sd0, sd1)
    sah = (sa0, sa1)
    srh = (sr0, sr1)

    def start_chunk(c):
        b = c & 1
        cbase = base + c * CHUNK
        idx_c = idx_all.at[pl.ds(c * CHUNK, CHUNK)]
        return (
            pltpu.async_copy(tpl_hbm.at[idx_c], yb_[b], sy[b]),
            pltpu.async_copy(x_hbm.at[pl.ds(cbase, CHUNK)], xb_[b], sx[b]),
        )

    pend = start_chunk(0)
    out_pend = [None, None]
    for c in range(NCHUNK):
        b = c & 1
        if c + 1 < NCHUNK:
            nxt = start_chunk(c + 1)
        for h in pend:
            h.wait()
        if out_pend[b] is not None:
            for h in out_pend[b]:
                h.wait()
        y_v = yb_[b]
        x_v = xb_[b]

        def grp_body(g, carry2):
            def dim_body(d, accs):
                acc_d, acc_a = accs
                rows = g * L + iota
                cols = (d + iota) & (INPUT_SIZE - 1)
                xv = plsc.load_gather(x_v, [rows, cols])
                ya = plsc.load_gather(y_v, [rows, cols])
                yc = plsc.load_gather(y_v, [rows, cols + INPUT_SIZE])
                s = 1.0 / (1.0 + jnp.exp(-xv))
                cmp = 1.0 - s
                da = s - ya
                db = cmp - yc
                acc_d = acc_d + da * da + db * db
                acc_a = acc_a + jnp.minimum(s, ya) + jnp.minimum(cmp, yc)
                return acc_d, acc_a

            acc_d, acc_a = lax.fori_loop(
                0, INPUT_SIZE, dim_body, (zero, zero), unroll=4)
            od[b][pl.ds(g * L, L)] = acc_d
            oa[b][pl.ds(g * L, L)] = acc_a
            orr[b][pl.ds(g * L, L)] = acc_a * (1.0 / INPUT_SIZE)
            return carry2

        lax.fori_loop(0, NGRP, grp_body, 0)
        cb = base + c * CHUNK
        out_pend[b] = (
            pltpu.async_copy(od[b], d_hbm.at[pl.ds(cb, CHUNK)], sdh[b]),
            pltpu.async_copy(oa[b], a_hbm.at[pl.ds(cb, CHUNK)], sah[b]),
            pltpu.async_copy(orr[b], r_hbm.at[pl.ds(cb, CHUNK)], srh[b]),
        )
        if c + 1 < NCHUNK:
            pend = nxt
    for hs in out_pend:
        if hs is not None:
            for h in hs:
                h.wait()


@functools.partial(
    pl.kernel,
    mesh=_mesh,
    compiler_params=pltpu.CompilerParams(needs_layout_passes=False),
    out_type=[
        jax.ShapeDtypeStruct((N_TOKENS,), jnp.float32),
        jax.ShapeDtypeStruct((N_TOKENS,), jnp.float32),
    ],
    scratch_types=[
        pltpu.VMEM((TPW,), jnp.int32),
        pltpu.VMEM((TPW,), jnp.float32),
        pltpu.VMEM((TPW,), jnp.float32),
        pltpu.VMEM((TPW,), jnp.float32),
        pltpu.VMEM((TPW,), jnp.float32),
        pltpu.SemaphoreType.DMA,
        pltpu.SemaphoreType.DMA,
        pltpu.SemaphoreType.DMA,
    ],
)
def _sc_tail(ms_hbm, a_hbm, dwo_hbm, sden_hbm, dwoo_hbm, sam_hbm,
             idx_all, av, dwov, sdenv, samv, s0, s1, s2):
    wid = lax.axis_index("s") * 2 + lax.axis_index("c")
    base = wid * TPW
    pltpu.sync_copy(ms_hbm.at[pl.ds(base, TPW)], idx_all)
    ha = pltpu.async_copy(a_hbm.at[pl.ds(base, TPW)], av, s0)
    hs = []
    for c in range(NCHUNK):
        idx_c = idx_all.at[pl.ds(c * CHUNK, CHUNK)]
        hs.append(pltpu.async_copy(
            dwo_hbm.at[idx_c], dwov.at[pl.ds(c * CHUNK, CHUNK)], s1))
        hs.append(pltpu.async_copy(
            sden_hbm.at[idx_c], sdenv.at[pl.ds(c * CHUNK, CHUNK)], s2))
    ha.wait()
    for h in hs:
        h.wait()

    def grp_body(g, carry):
        sl = pl.ds(g * L, L)
        samv[sl] = av[sl] / sdenv[sl]
        return carry

    lax.fori_loop(0, TPW // L, grp_body, 0)
    h1 = pltpu.async_copy(dwov, dwoo_hbm.at[pl.ds(base, TPW)], s1)
    h2 = pltpu.async_copy(samv, sam_hbm.at[pl.ds(base, TPW)], s2)
    h1.wait()
    h2.wait()


@jax.jit
def kernel(x, ms, templates):
    msi = ms.astype(jnp.int32)
    stats = _stats_call(templates)
    dwo, sden = _tbl_call(templates, stats)
    d_arr, a_arr, r_arr = _sc_main(x, msi, templates)
    dwo_arr, sam_arr = _sc_tail(msi, a_arr, dwo.reshape(K), sden.reshape(K))
    return jnp.stack([d_arr, dwo_arr, sam_arr, r_arr], axis=-1)


# speedup vs baseline: 2.8342x; 1.0202x over previous
import functools

import jax
import jax.numpy as jnp
from jax import lax
from jax.experimental import pallas as pl
from jax.experimental.pallas import tpu as pltpu
from jax.experimental.pallas import tpu_sc as plsc

ALPHA = 0.05
N_TOKENS = 16384
INPUT_SIZE = 128
K = 8192
DIM = 2 * INPUT_SIZE

NW = 32
TPW = N_TOKENS // NW
CHUNK = 64
NCHUNK = TPW // CHUNK
L = 16
NGRP = CHUNK // L

_BLK = 1024
_GRID = K // _BLK


def _stats_body(t_ref, o_ref):
    i = pl.program_id(0)
    blk = t_ref[...]
    ps = jnp.sum(blk, axis=0, keepdims=True) * (-2.0 / K)
    pq = jnp.sum(blk * blk) * (1.0 / K)

    @pl.when(i == 0)
    def _():
        o_ref[...] = jnp.zeros((2, DIM), jnp.float32)

    o_ref[0:1, :] += ps
    o_ref[1:2, :] += jnp.full((1, DIM), pq, jnp.float32)


_stats_call = pl.pallas_call(
    _stats_body,
    grid=(_GRID,),
    in_specs=[pl.BlockSpec((_BLK, DIM), lambda i: (i, 0))],
    out_specs=pl.BlockSpec((2, DIM), lambda i: (0, 0)),
    out_shape=jax.ShapeDtypeStruct((2, DIM), jnp.float32),
)


def _tbl_body(t_ref, st_ref, dwo_ref, sden_ref):
    blk = t_ref[...]
    tsn = st_ref[0:1, :]
    c0 = st_ref[1, 0]
    s2sv = jnp.sum(blk * blk + blk * tsn, axis=1)
    s1 = jnp.sum(blk, axis=1)
    dwo_ref[...] = (c0 + s2sv).reshape(_BLK // INPUT_SIZE, INPUT_SIZE)
    sden_ref[...] = (ALPHA + s1).reshape(_BLK // INPUT_SIZE, INPUT_SIZE)


_tbl_call = pl.pallas_call(
    _tbl_body,
    grid=(_GRID,),
    in_specs=[
        pl.BlockSpec((_BLK, DIM), lambda i: (i, 0)),
        pl.BlockSpec((2, DIM), lambda i: (0, 0)),
    ],
    out_specs=[
        pl.BlockSpec((_BLK // INPUT_SIZE, INPUT_SIZE), lambda i: (i, 0)),
        pl.BlockSpec((_BLK // INPUT_SIZE, INPUT_SIZE), lambda i: (i, 0)),
    ],
    out_shape=[
        jax.ShapeDtypeStruct((K // INPUT_SIZE, INPUT_SIZE), jnp.float32),
        jax.ShapeDtypeStruct((K // INPUT_SIZE, INPUT_SIZE), jnp.float32),
    ],
)

_mesh = plsc.VectorSubcoreMesh(core_axis_name="c", subcore_axis_name="s")


@functools.partial(
    pl.kernel,
    mesh=_mesh,
    compiler_params=pltpu.CompilerParams(needs_layout_passes=False),
    out_type=[
        jax.ShapeDtypeStruct((N_TOKENS,), jnp.float32),
        jax.ShapeDtypeStruct((N_TOKENS,), jnp.float32),
        jax.ShapeDtypeStruct((N_TOKENS,), jnp.float32),
    ],
    scratch_types=[
        pltpu.VMEM((TPW,), jnp.int32),
        pltpu.VMEM((CHUNK, DIM), jnp.float32),
        pltpu.VMEM((CHUNK, DIM), jnp.float32),
        pltpu.VMEM((CHUNK, INPUT_SIZE), jnp.float32),
        pltpu.VMEM((CHUNK, INPUT_SIZE), jnp.float32),
        pltpu.VMEM((CHUNK,), jnp.float32),
        pltpu.VMEM((CHUNK,), jnp.float32),
        pltpu.VMEM((CHUNK,), jnp.float32),
        pltpu.VMEM((CHUNK,), jnp.float32),
        pltpu.VMEM((CHUNK,), jnp.float32),
        pltpu.VMEM((CHUNK,), jnp.float32),
    ] + [pltpu.SemaphoreType.DMA] * 10,
)
def _sc_main(x_hbm, ms_hbm, tpl_hbm, d_hbm, a_hbm, r_hbm,
             idx_all, y0, y1, x0, x1, od0, od1, oa0, oa1, or0, or1,
             sy0, sy1, sx0, sx1, sd0, sd1, sa0, sa1, sr0, sr1):
    wid = lax.axis_index("s") * 2 + lax.axis_index("c")
    base = wid * TPW
    pltpu.sync_copy(ms_hbm.at[pl.ds(base, TPW)], idx_all)
    iota = lax.iota(jnp.int32, L)
    zero = jnp.zeros((L,), jnp.float32)
    yb_ = (y0, y1)
    xb_ = (x0, x1)
    od = (od0, od1)
    oa = (oa0, oa1)
    orr = (or0, or1)
    sy = (sy0, sy1)
    sx = (sx0, sx1)
    sdh = (sd0, sd1)
    sah = (sa0, sa1)
    srh = (sr0, sr1)

    def start_chunk(c):
        b = c & 1
        cbase = base + c * CHUNK
        idx_c = idx_all.at[pl.ds(c * CHUNK, CHUNK)]
        return (
            pltpu.async_copy(tpl_hbm.at[idx_c], yb_[b], sy[b]),
            pltpu.async_copy(x_hbm.at[pl.ds(cbase, CHUNK)], xb_[b], sx[b]),
        )

    pend = start_chunk(0)
    out_pend = [None, None]
    for c in range(NCHUNK):
        b = c & 1
        if c + 1 < NCHUNK:
            nxt = start_chunk(c + 1)
        for h in pend:
            h.wait()
        if out_pend[b] is not None:
            for h in out_pend[b]:
                h.wait()
        y_v = yb_[b]
        x_v = xb_[b]

        def grp_body(g, carry2):
            rows = g * L + iota

            def dim_body(d, accs):
                acc_d, acc_a = accs
                cols = (d + iota) & (INPUT_SIZE - 1)
                xv = plsc.load_gather(x_v, [rows, cols])
                ya = plsc.load_gather(y_v, [rows, cols])
                yc = plsc.load_gather(y_v, [rows, cols + INPUT_SIZE])
                s = 1.0 / (1.0 + jnp.exp(-xv))
                cmp = 1.0 - s
                da = s - ya
                db = cmp - yc
                acc_d = acc_d + da * da + db * db
                acc_a = acc_a + jnp.minimum(s, ya) + jnp.minimum(cmp, yc)
                return acc_d, acc_a

            acc_d, acc_a = lax.fori_loop(
                0, INPUT_SIZE, dim_body, (zero, zero), unroll=2)
            od[b][pl.ds(g * L, L)] = acc_d
            oa[b][pl.ds(g * L, L)] = acc_a
            orr[b][pl.ds(g * L, L)] = acc_a * (1.0 / INPUT_SIZE)
            return carry2

        lax.fori_loop(0, NGRP, grp_body, 0)
        cb = base + c * CHUNK
        out_pend[b] = (
            pltpu.async_copy(od[b], d_hbm.at[pl.ds(cb, CHUNK)], sdh[b]),
            pltpu.async_copy(oa[b], a_hbm.at[pl.ds(cb, CHUNK)], sah[b]),
            pltpu.async_copy(orr[b], r_hbm.at[pl.ds(cb, CHUNK)], srh[b]),
        )
        if c + 1 < NCHUNK:
            pend = nxt
    for hs in out_pend:
        if hs is not None:
            for h in hs:
                h.wait()


@functools.partial(
    pl.kernel,
    mesh=_mesh,
    compiler_params=pltpu.CompilerParams(needs_layout_passes=False),
    out_type=[
        jax.ShapeDtypeStruct((N_TOKENS,), jnp.float32),
        jax.ShapeDtypeStruct((N_TOKENS,), jnp.float32),
    ],
    scratch_types=[
        pltpu.VMEM((TPW,), jnp.int32),
        pltpu.VMEM((TPW,), jnp.float32),
        pltpu.VMEM((TPW,), jnp.float32),
        pltpu.VMEM((TPW,), jnp.float32),
        pltpu.VMEM((TPW,), jnp.float32),
        pltpu.SemaphoreType.DMA,
        pltpu.SemaphoreType.DMA,
        pltpu.SemaphoreType.DMA,
    ],
)
def _sc_tail(ms_hbm, a_hbm, dwo_hbm, sden_hbm, dwoo_hbm, sam_hbm,
             idx_all, av, dwov, sdenv, samv, s0, s1, s2):
    wid = lax.axis_index("s") * 2 + lax.axis_index("c")
    base = wid * TPW
    pltpu.sync_copy(ms_hbm.at[pl.ds(base, TPW)], idx_all)
    ha = pltpu.async_copy(a_hbm.at[pl.ds(base, TPW)], av, s0)
    hs = []
    for c in range(NCHUNK):
        idx_c = idx_all.at[pl.ds(c * CHUNK, CHUNK)]
        hs.append(pltpu.async_copy(
            dwo_hbm.at[idx_c], dwov.at[pl.ds(c * CHUNK, CHUNK)], s1))
        hs.append(pltpu.async_copy(
            sden_hbm.at[idx_c], sdenv.at[pl.ds(c * CHUNK, CHUNK)], s2))
    ha.wait()
    for h in hs:
        h.wait()

    def grp_body(g, carry):
        sl = pl.ds(g * L, L)
        samv[sl] = av[sl] / sdenv[sl]
        return carry

    lax.fori_loop(0, TPW // L, grp_body, 0)
    h1 = pltpu.async_copy(dwov, dwoo_hbm.at[pl.ds(base, TPW)], s1)
    h2 = pltpu.async_copy(samv, sam_hbm.at[pl.ds(base, TPW)], s2)
    h1.wait()
    h2.wait()


@jax.jit
def kernel(x, ms, templates):
    msi = ms.astype(jnp.int32)
    stats = _stats_call(templates)
    dwo, sden = _tbl_call(templates, stats)
    d_arr, a_arr, r_arr = _sc_main(x, msi, templates)
    dwo_arr, sam_arr = _sc_tail(msi, a_arr, dwo.reshape(K), sden.reshape(K))
    return jnp.stack([d_arr, dwo_arr, sam_arr, r_arr], axis=-1)


# speedup vs baseline: 2.8479x; 1.0048x over previous
import functools

import jax
import jax.numpy as jnp
from jax import lax
from jax.experimental import pallas as pl
from jax.experimental.pallas import tpu as pltpu
from jax.experimental.pallas import tpu_sc as plsc

ALPHA = 0.05
N_TOKENS = 16384
INPUT_SIZE = 128
K = 8192
DIM = 2 * INPUT_SIZE

NW = 32
TPW = N_TOKENS // NW
CHUNK = 64
NCHUNK = TPW // CHUNK
L = 16
NGRP = CHUNK // L

_BLK = 1024
_GRID = K // _BLK


def _stats_body(t_ref, o_ref):
    i = pl.program_id(0)
    blk = t_ref[...]
    ps = jnp.sum(blk, axis=0, keepdims=True) * (-2.0 / K)
    pq = jnp.sum(blk * blk) * (1.0 / K)

    @pl.when(i == 0)
    def _():
        o_ref[...] = jnp.zeros((2, DIM), jnp.float32)

    o_ref[0:1, :] += ps
    o_ref[1:2, :] += jnp.full((1, DIM), pq, jnp.float32)


_stats_call = pl.pallas_call(
    _stats_body,
    grid=(_GRID,),
    in_specs=[pl.BlockSpec((_BLK, DIM), lambda i: (i, 0))],
    out_specs=pl.BlockSpec((2, DIM), lambda i: (0, 0)),
    out_shape=jax.ShapeDtypeStruct((2, DIM), jnp.float32),
)


def _tbl_body(t_ref, st_ref, dwo_ref, sden_ref):
    blk = t_ref[...]
    tsn = st_ref[0:1, :]
    c0 = st_ref[1, 0]
    s2sv = jnp.sum(blk * blk + blk * tsn, axis=1)
    s1 = jnp.sum(blk, axis=1)
    dwo_ref[...] = (c0 + s2sv).reshape(_BLK // INPUT_SIZE, INPUT_SIZE)
    sden_ref[...] = (ALPHA + s1).reshape(_BLK // INPUT_SIZE, INPUT_SIZE)


_tbl_call = pl.pallas_call(
    _tbl_body,
    grid=(_GRID,),
    in_specs=[
        pl.BlockSpec((_BLK, DIM), lambda i: (i, 0)),
        pl.BlockSpec((2, DIM), lambda i: (0, 0)),
    ],
    out_specs=[
        pl.BlockSpec((_BLK // INPUT_SIZE, INPUT_SIZE), lambda i: (i, 0)),
        pl.BlockSpec((_BLK // INPUT_SIZE, INPUT_SIZE), lambda i: (i, 0)),
    ],
    out_shape=[
        jax.ShapeDtypeStruct((K // INPUT_SIZE, INPUT_SIZE), jnp.float32),
        jax.ShapeDtypeStruct((K // INPUT_SIZE, INPUT_SIZE), jnp.float32),
    ],
)

_mesh = plsc.VectorSubcoreMesh(core_axis_name="c", subcore_axis_name="s")


@functools.partial(
    pl.kernel,
    mesh=_mesh,
    compiler_params=pltpu.CompilerParams(needs_layout_passes=False),
    out_type=[
        jax.ShapeDtypeStruct((N_TOKENS,), jnp.float32),
        jax.ShapeDtypeStruct((N_TOKENS,), jnp.float32),
        jax.ShapeDtypeStruct((N_TOKENS,), jnp.float32),
    ],
    scratch_types=[
        pltpu.VMEM((TPW,), jnp.int32),
        pltpu.VMEM((CHUNK, DIM), jnp.float32),
        pltpu.VMEM((CHUNK, DIM), jnp.float32),
        pltpu.VMEM((CHUNK, INPUT_SIZE), jnp.float32),
        pltpu.VMEM((CHUNK, INPUT_SIZE), jnp.float32),
        pltpu.VMEM((CHUNK,), jnp.float32),
        pltpu.VMEM((CHUNK,), jnp.float32),
        pltpu.VMEM((CHUNK,), jnp.float32),
        pltpu.VMEM((CHUNK,), jnp.float32),
        pltpu.VMEM((CHUNK,), jnp.float32),
        pltpu.VMEM((CHUNK,), jnp.float32),
    ] + [pltpu.SemaphoreType.DMA] * 10,
)
def _sc_main(x_hbm, ms_hbm, tpl_hbm, d_hbm, a_hbm, r_hbm,
             idx_all, y0, y1, x0, x1, od0, od1, oa0, oa1, or0, or1,
             sy0, sy1, sx0, sx1, sd0, sd1, sa0, sa1, sr0, sr1):
    wid = lax.axis_index("s") * 2 + lax.axis_index("c")
    base = wid * TPW
    pltpu.sync_copy(ms_hbm.at[pl.ds(base, TPW)], idx_all)
    iota = lax.iota(jnp.int32, L)
    zero = jnp.zeros((L,), jnp.float32)
    yb_ = (y0, y1)
    xb_ = (x0, x1)
    od = (od0, od1)
    oa = (oa0, oa1)
    orr = (or0, or1)
    sy = (sy0, sy1)
    sx = (sx0, sx1)
    sdh = (sd0, sd1)
    sah = (sa0, sa1)
    srh = (sr0, sr1)

    def start_chunk(c):
        b = c & 1
        cbase = base + c * CHUNK
        idx_c = idx_all.at[pl.ds(c * CHUNK, CHUNK)]
        return (
            pltpu.async_copy(tpl_hbm.at[idx_c], yb_[b], sy[b]),
            pltpu.async_copy(x_hbm.at[pl.ds(cbase, CHUNK)], xb_[b], sx[b]),
        )

    pend = start_chunk(0)
    out_pend = [None, None]
    for c in range(NCHUNK):
        b = c & 1
        if c + 1 < NCHUNK:
            nxt = start_chunk(c + 1)
        for h in pend:
            h.wait()
        if out_pend[b] is not None:
            for h in out_pend[b]:
                h.wait()
        y_v = yb_[b]
        x_v = xb_[b]

        def grp_body(g, carry2):
            rows = g * L + iota

            def dim_body(d, accs):
                acc_d, acc_a = accs
                cols = (d + iota) & (INPUT_SIZE - 1)
                xv = plsc.load_gather(x_v, [rows, cols])
                ya = plsc.load_gather(y_v, [rows, cols])
                yc = plsc.load_gather(y_v, [rows, cols + INPUT_SIZE])
                cmp = 1.0 / (1.0 + jnp.exp(xv))
                s = 1.0 - cmp
                da = s - ya
                db = cmp - yc
                acc_d = acc_d + da * da + db * db
                acc_a = acc_a + jnp.minimum(s, ya) + jnp.minimum(cmp, yc)
                return acc_d, acc_a

            acc_d, acc_a = lax.fori_loop(
                0, INPUT_SIZE, dim_body, (zero, zero), unroll=2)
            od[b][pl.ds(g * L, L)] = acc_d
            oa[b][pl.ds(g * L, L)] = acc_a
            orr[b][pl.ds(g * L, L)] = acc_a * (1.0 / INPUT_SIZE)
            return carry2

        lax.fori_loop(0, NGRP, grp_body, 0)
        cb = base + c * CHUNK
        out_pend[b] = (
            pltpu.async_copy(od[b], d_hbm.at[pl.ds(cb, CHUNK)], sdh[b]),
            pltpu.async_copy(oa[b], a_hbm.at[pl.ds(cb, CHUNK)], sah[b]),
            pltpu.async_copy(orr[b], r_hbm.at[pl.ds(cb, CHUNK)], srh[b]),
        )
        if c + 1 < NCHUNK:
            pend = nxt
    for hs in out_pend:
        if hs is not None:
            for h in hs:
                h.wait()


@functools.partial(
    pl.kernel,
    mesh=_mesh,
    compiler_params=pltpu.CompilerParams(needs_layout_passes=False),
    out_type=[
        jax.ShapeDtypeStruct((N_TOKENS,), jnp.float32),
        jax.ShapeDtypeStruct((N_TOKENS,), jnp.float32),
    ],
    scratch_types=[
        pltpu.VMEM((TPW,), jnp.int32),
        pltpu.VMEM((TPW,), jnp.float32),
        pltpu.VMEM((TPW,), jnp.float32),
        pltpu.VMEM((TPW,), jnp.float32),
        pltpu.VMEM((TPW,), jnp.float32),
        pltpu.SemaphoreType.DMA,
        pltpu.SemaphoreType.DMA,
        pltpu.SemaphoreType.DMA,
    ],
)
def _sc_tail(ms_hbm, a_hbm, dwo_hbm, sden_hbm, dwoo_hbm, sam_hbm,
             idx_all, av, dwov, sdenv, samv, s0, s1, s2):
    wid = lax.axis_index("s") * 2 + lax.axis_index("c")
    base = wid * TPW
    pltpu.sync_copy(ms_hbm.at[pl.ds(base, TPW)], idx_all)
    ha = pltpu.async_copy(a_hbm.at[pl.ds(base, TPW)], av, s0)
    hs = []
    for c in range(NCHUNK):
        idx_c = idx_all.at[pl.ds(c * CHUNK, CHUNK)]
        hs.append(pltpu.async_copy(
            dwo_hbm.at[idx_c], dwov.at[pl.ds(c * CHUNK, CHUNK)], s1))
        hs.append(pltpu.async_copy(
            sden_hbm.at[idx_c], sdenv.at[pl.ds(c * CHUNK, CHUNK)], s2))
    ha.wait()
    for h in hs:
        h.wait()

    def grp_body(g, carry):
        sl = pl.ds(g * L, L)
        samv[sl] = av[sl] / sdenv[sl]
        return carry

    lax.fori_loop(0, TPW // L, grp_body, 0)
    h1 = pltpu.async_copy(dwov, dwoo_hbm.at[pl.ds(base, TPW)], s1)
    h2 = pltpu.async_copy(samv, sam_hbm.at[pl.ds(base, TPW)], s2)
    h1.wait()
    h2.wait()


@jax.jit
def kernel(x, ms, templates):
    msi = ms.astype(jnp.int32)
    stats = _stats_call(templates)
    dwo, sden = _tbl_call(templates, stats)
    d_arr, a_arr, r_arr = _sc_main(x, msi, templates)
    dwo_arr, sam_arr = _sc_tail(msi, a_arr, dwo.reshape(K), sden.reshape(K))
    return jnp.stack([d_arr, dwo_arr, sam_arr, r_arr], axis=-1)


# speedup vs baseline: 2.8544x; 1.0023x over previous
import functools

import jax
import jax.numpy as jnp
from jax import lax
from jax.experimental import pallas as pl
from jax.experimental.pallas import tpu as pltpu
from jax.experimental.pallas import tpu_sc as plsc

ALPHA = 0.05
N_TOKENS = 16384
INPUT_SIZE = 128
K = 8192
DIM = 2 * INPUT_SIZE

NW = 32
TPW = N_TOKENS // NW
CHUNK = 64
NCHUNK = TPW // CHUNK
L = 16
NGRP = CHUNK // L

_BLK = 1024
_GRID = K // _BLK


def _stats_body(t_ref, o_ref):
    i = pl.program_id(0)
    blk = t_ref[...]
    ps = jnp.sum(blk, axis=0, keepdims=True) * (-2.0 / K)
    pq = jnp.sum(blk * blk) * (1.0 / K)

    @pl.when(i == 0)
    def _():
        o_ref[...] = jnp.zeros((2, DIM), jnp.float32)

    o_ref[0:1, :] += ps
    o_ref[1:2, :] += jnp.full((1, DIM), pq, jnp.float32)


_stats_call = pl.pallas_call(
    _stats_body,
    grid=(_GRID,),
    in_specs=[pl.BlockSpec((_BLK, DIM), lambda i: (i, 0))],
    out_specs=pl.BlockSpec((2, DIM), lambda i: (0, 0)),
    out_shape=jax.ShapeDtypeStruct((2, DIM), jnp.float32),
)


def _tbl_body(t_ref, st_ref, dwo_ref, sden_ref):
    blk = t_ref[...]
    tsn = st_ref[0:1, :]
    c0 = st_ref[1, 0]
    s2sv = jnp.sum(blk * blk + blk * tsn, axis=1)
    s1 = jnp.sum(blk, axis=1)
    dwo_ref[...] = (c0 + s2sv).reshape(_BLK // INPUT_SIZE, INPUT_SIZE)
    sden_ref[...] = (ALPHA + s1).reshape(_BLK // INPUT_SIZE, INPUT_SIZE)


_tbl_call = pl.pallas_call(
    _tbl_body,
    grid=(_GRID,),
    in_specs=[
        pl.BlockSpec((_BLK, DIM), lambda i: (i, 0)),
        pl.BlockSpec((2, DIM), lambda i: (0, 0)),
    ],
    out_specs=[
        pl.BlockSpec((_BLK // INPUT_SIZE, INPUT_SIZE), lambda i: (i, 0)),
        pl.BlockSpec((_BLK // INPUT_SIZE, INPUT_SIZE), lambda i: (i, 0)),
    ],
    out_shape=[
        jax.ShapeDtypeStruct((K // INPUT_SIZE, INPUT_SIZE), jnp.float32),
        jax.ShapeDtypeStruct((K // INPUT_SIZE, INPUT_SIZE), jnp.float32),
    ],
)

_mesh = plsc.VectorSubcoreMesh(core_axis_name="c", subcore_axis_name="s")


@functools.partial(
    pl.kernel,
    mesh=_mesh,
    compiler_params=pltpu.CompilerParams(needs_layout_passes=False),
    out_type=[
        jax.ShapeDtypeStruct((N_TOKENS,), jnp.float32),
        jax.ShapeDtypeStruct((N_TOKENS,), jnp.float32),
        jax.ShapeDtypeStruct((N_TOKENS,), jnp.float32),
    ],
    scratch_types=[
        pltpu.VMEM((TPW,), jnp.int32),
        pltpu.VMEM((CHUNK, DIM), jnp.float32),
        pltpu.VMEM((CHUNK, DIM), jnp.float32),
        pltpu.VMEM((CHUNK, INPUT_SIZE), jnp.float32),
        pltpu.VMEM((CHUNK, INPUT_SIZE), jnp.float32),
        pltpu.VMEM((CHUNK,), jnp.float32),
        pltpu.VMEM((CHUNK,), jnp.float32),
        pltpu.VMEM((CHUNK,), jnp.float32),
        pltpu.VMEM((CHUNK,), jnp.float32),
        pltpu.VMEM((CHUNK,), jnp.float32),
        pltpu.VMEM((CHUNK,), jnp.float32),
    ] + [pltpu.SemaphoreType.DMA] * 10,
)
def _sc_main(x_hbm, ms_hbm, tpl_hbm, d_hbm, a_hbm, r_hbm,
             idx_all, y0, y1, x0, x1, od0, od1, oa0, oa1, or0, or1,
             sy0, sy1, sx0, sx1, sd0, sd1, sa0, sa1, sr0, sr1):
    wid = lax.axis_index("s") * 2 + lax.axis_index("c")
    base = wid * TPW
    pltpu.sync_copy(ms_hbm.at[pl.ds(base, TPW)], idx_all)
    iota = lax.iota(jnp.int32, L)
    zero = jnp.zeros((L,), jnp.float32)
    yb_ = (y0, y1)
    xb_ = (x0, x1)
    od = (od0, od1)
    oa = (oa0, oa1)
    orr = (or0, or1)
    sy = (sy0, sy1)
    sx = (sx0, sx1)
    sdh = (sd0, sd1)
    sah = (sa0, sa1)
    srh = (sr0, sr1)

    def start_chunk(c):
        b = c & 1
        cbase = base + c * CHUNK
        idx_c = idx_all.at[pl.ds(c * CHUNK, CHUNK)]
        return (
            pltpu.async_copy(tpl_hbm.at[idx_c], yb_[b], sy[b]),
            pltpu.async_copy(x_hbm.at[pl.ds(cbase, CHUNK)], xb_[b], sx[b]),
        )

    pend = start_chunk(0)
    out_pend = [None, None]
    for c in range(NCHUNK):
        b = c & 1
        if c + 1 < NCHUNK:
            nxt = start_chunk(c + 1)
        for h in pend:
            h.wait()
        if out_pend[b] is not None:
            for h in out_pend[b]:
                h.wait()
        y_v = yb_[b]
        x_v = xb_[b]

        def grp_body(g, carry2):
            rows = g * L + iota

            def dim_body(d, accs):
                acc_d, acc_a = accs
                cols = (d + iota) & (INPUT_SIZE - 1)
                xv = plsc.load_gather(x_v, [rows, cols])
                ya = plsc.load_gather(y_v, [rows, cols])
                yc = plsc.load_gather(y_v, [rows, cols + INPUT_SIZE])
                cmp = 1.0 / (1.0 + jnp.exp(xv))
                s = 1.0 - cmp
                da = s - ya
                db = cmp - yc
                acc_d = acc_d + da * da + db * db
                acc_a = acc_a + jnp.minimum(s, ya) + jnp.minimum(cmp, yc)
                return acc_d, acc_a

            acc_d, acc_a = lax.fori_loop(
                0, INPUT_SIZE, dim_body, (zero, zero), unroll=2)
            od[b][pl.ds(g * L, L)] = acc_d
            oa[b][pl.ds(g * L, L)] = acc_a
            orr[b][pl.ds(g * L, L)] = acc_a * (1.0 / INPUT_SIZE)
            return carry2

        lax.fori_loop(0, NGRP, grp_body, 0)
        cb = base + c * CHUNK
        out_pend[b] = (
            pltpu.async_copy(od[b], d_hbm.at[pl.ds(cb, CHUNK)], sdh[b]),
            pltpu.async_copy(oa[b], a_hbm.at[pl.ds(cb, CHUNK)], sah[b]),
            pltpu.async_copy(orr[b], r_hbm.at[pl.ds(cb, CHUNK)], srh[b]),
        )
        if c + 1 < NCHUNK:
            pend = nxt
    for hs in out_pend:
        if hs is not None:
            for h in hs:
                h.wait()


@functools.partial(
    pl.kernel,
    mesh=_mesh,
    compiler_params=pltpu.CompilerParams(needs_layout_passes=False),
    out_type=[
        jax.ShapeDtypeStruct((N_TOKENS,), jnp.float32),
        jax.ShapeDtypeStruct((N_TOKENS,), jnp.float32),
    ],
    scratch_types=[
        pltpu.VMEM((TPW,), jnp.int32),
        pltpu.VMEM((TPW,), jnp.float32),
        pltpu.VMEM((TPW,), jnp.float32),
        pltpu.VMEM((TPW,), jnp.float32),
        pltpu.VMEM((TPW,), jnp.float32),
        pltpu.SemaphoreType.DMA,
        pltpu.SemaphoreType.DMA,
        pltpu.SemaphoreType.DMA,
    ],
)
def _sc_tail(ms_hbm, a_hbm, dwo_hbm, sden_hbm, dwoo_hbm, sam_hbm,
             idx_all, av, dwov, sdenv, samv, s0, s1, s2):
    wid = lax.axis_index("s") * 2 + lax.axis_index("c")
    base = wid * TPW
    pltpu.sync_copy(ms_hbm.at[pl.ds(base, TPW)], idx_all)
    ha = pltpu.async_copy(a_hbm.at[pl.ds(base, TPW)], av, s0)
    hs = []
    for c in range(NCHUNK):
        idx_c = idx_all.at[pl.ds(c * CHUNK, CHUNK)]
        hs.append(pltpu.async_copy(
            dwo_hbm.at[idx_c], dwov.at[pl.ds(c * CHUNK, CHUNK)], s1))
        hs.append(pltpu.async_copy(
            sden_hbm.at[idx_c], sdenv.at[pl.ds(c * CHUNK, CHUNK)], s2))
    ha.wait()
    for h in hs:
        h.wait()

    def grp_body(g, carry):
        sl = pl.ds(g * L, L)
        samv[sl] = av[sl] / sdenv[sl]
        return carry

    lax.fori_loop(0, TPW // L, grp_body, 0)
    h1 = pltpu.async_copy(dwov, dwoo_hbm.at[pl.ds(base, TPW)], s1)
    h2 = pltpu.async_copy(samv, sam_hbm.at[pl.ds(base, TPW)], s2)
    h1.wait()
    h2.wait()


@jax.jit
def kernel(x, ms, templates):
    msi = ms.astype(jnp.int32)
    d_arr, a_arr, r_arr = _sc_main(x, msi, templates)
    stats = _stats_call(templates)
    dwo, sden = _tbl_call(templates, stats)
    dwo_arr, sam_arr = _sc_tail(msi, a_arr, dwo.reshape(K), sden.reshape(K))
    return jnp.stack([d_arr, dwo_arr, sam_arr, r_arr], axis=-1)
